# Initial kernel scaffold; baseline (speedup 1.0000x reference)
import jax
import jax.numpy as jnp
from jax.experimental import pallas as pl


def kernel(x, W_q, W_k, W_v, W_o, W_ck, W_cv, W_g, b_g):
    raise NotImplementedError("write your pallas kernel here")


# speedup vs baseline: 101.3705x; 101.3705x over previous
import math

import jax
import jax.numpy as jnp
from jax.experimental import pallas as pl
from jax.experimental.pallas import tpu as pltpu

B, T, D = 1, 2048, 768
H, HD = 12, 64
G, R, WIN = 16, 4, 64
NB = T // G
SCALE = math.sqrt(HD)
HALF = WIN // 2
TQ = 256
NEG_TOPK = -1.0e30
NEG_TAKEN = -3.0e38

_INTERPRET = False


def _safe_sm(s):
    m = jnp.max(s, axis=-1, keepdims=True)
    m = jnp.where(m > -jnp.inf, m, 0.0)
    e = jnp.exp(s - m)
    z = jnp.sum(e, axis=-1, keepdims=True)
    return e / jnp.where(z == 0.0, 1.0, z)


def _qkv_gates_kernel(x_ref, w_ref, wg_ref, bg_ref, qkv_ref, g_ref):
    xb = x_ref[...]
    qkv = jax.lax.dot_general(xb, w_ref[...], (((1,), (0,)), ((), ())),
                              preferred_element_type=jnp.float32)
    qkv_ref[...] = qkv
    qm = qkv[:, 0:HD]
    for h in range(1, H):
        qm = qm + qkv[:, h * HD:(h + 1) * HD]
    qm = qm * (1.0 / H)
    glog = jax.lax.dot_general(qm, wg_ref[...], (((1,), (1,)), ((), ())),
                               preferred_element_type=jnp.float32)
    glog = glog + bg_ref[...]
    m = jnp.max(glog, axis=-1, keepdims=True)
    e = jnp.exp(glog - m)
    g_ref[...] = e / jnp.sum(e, axis=-1, keepdims=True)


def _attn_kernel(q_ref, k_ref, v_ref, g_ref, wck_ref, wcv_ref, out_ref,
                 kc_s, vc_s):
    qc = pl.program_id(1)

    @pl.when(qc == 0)
    def _compress():
        k3 = k_ref[0].reshape(NB, G, HD)
        v3 = v_ref[0].reshape(NB, G, HD)
        kc = jnp.zeros((NB, HD), jnp.float32)
        vc = jnp.zeros((NB, HD), jnp.float32)
        for g in range(G):
            wck_g = wck_ref[:, g * HD:(g + 1) * HD]
            wcv_g = wcv_ref[:, g * HD:(g + 1) * HD]
            kc = kc + jax.lax.dot_general(k3[:, g, :], wck_g,
                                          (((1,), (1,)), ((), ())),
                                          preferred_element_type=jnp.float32)
            vc = vc + jax.lax.dot_general(v3[:, g, :], wcv_g,
                                          (((1,), (1,)), ((), ())),
                                          preferred_element_type=jnp.float32)
        kc_s[...] = kc
        vc_s[...] = vc

    q = q_ref[0]
    sc = jax.lax.dot_general(q, kc_s[...], (((1,), (1,)), ((), ())),
                             preferred_element_type=jnp.float32) * (1.0 / SCALE)
    bi = jax.lax.broadcasted_iota(jnp.int32, (TQ, NB), 1)
    ri = jax.lax.broadcasted_iota(jnp.int32, (TQ, NB), 0) + qc * TQ
    valid = ri >= (bi + 1) * G
    out_c = jax.lax.dot_general(_safe_sm(jnp.where(valid, sc, -jnp.inf)),
                                vc_s[...], (((1,), (0,)), ((), ())),
                                preferred_element_type=jnp.float32)
    s = jnp.where(valid, sc, NEG_TOPK)
    idxs = []
    for _ in range(R):
        m = jnp.max(s, axis=-1, keepdims=True)
        idx = jnp.min(jnp.where(s == m, bi, NB), axis=-1, keepdims=True)
        idxs.append(idx)
        s = jnp.where(bi == idx, NEG_TAKEN, s)
    ti = jax.lax.broadcasted_iota(jnp.int32, (TQ, T), 1)
    tr = jax.lax.broadcasted_iota(jnp.int32, (TQ, T), 0) + qc * TQ
    bcol = ti // G
    selm = (bcol == idxs[0]) | (bcol == idxs[1]) | (bcol == idxs[2]) | (bcol == idxs[3])
    causal = ti <= tr
    win = causal & (ti >= tr - HALF)
    S = jax.lax.dot_general(q, k_ref[0], (((1,), (1,)), ((), ())),
                            preferred_element_type=jnp.float32) * (1.0 / SCALE)
    P_s = _safe_sm(jnp.where(selm & causal, S, -jnp.inf))
    P_l = _safe_sm(jnp.where(win, S, -jnp.inf))
    g = g_ref[...]
    alpha = g[:, 0:1]
    beta = g[:, 1:2]
    gamma = g[:, 2:3]
    P = beta * P_s + gamma * P_l
    out_sl = jax.lax.dot_general(P, v_ref[0], (((1,), (0,)), ((), ())),
                                 preferred_element_type=jnp.float32)
    out_ref[0] = alpha * out_c + out_sl


def _proj_kernel(x_ref, w_ref, o_ref):
    o_ref[...] = jax.lax.dot_general(x_ref[...], w_ref[...],
                                     (((1,), (1,)), ((), ())),
                                     preferred_element_type=jnp.float32)


def kernel(x, W_q, W_k, W_v, W_o, W_ck, W_cv, W_g, b_g):
    x2d = x.reshape(T, D)
    W_qkv = jnp.concatenate([W_q, W_k, W_v], axis=0).T
    Wg_pad = jnp.zeros((128, HD), jnp.float32).at[:3].set(W_g)
    bg_pad = jnp.full((1, 128), -1.0e30, jnp.float32).at[0, :3].set(b_g)

    qkv, gates = pl.pallas_call(
        _qkv_gates_kernel,
        grid=(T // TQ,),
        in_specs=[
            pl.BlockSpec((TQ, D), lambda i: (i, 0)),
            pl.BlockSpec((D, 3 * D), lambda i: (0, 0)),
            pl.BlockSpec((128, HD), lambda i: (0, 0)),
            pl.BlockSpec((1, 128), lambda i: (0, 0)),
        ],
        out_specs=[
            pl.BlockSpec((TQ, 3 * D), lambda i: (i, 0)),
            pl.BlockSpec((TQ, 128), lambda i: (i, 0)),
        ],
        out_shape=[
            jax.ShapeDtypeStruct((T, 3 * D), jnp.float32),
            jax.ShapeDtypeStruct((T, 128), jnp.float32),
        ],
        interpret=_INTERPRET,
    )(x2d, W_qkv, Wg_pad, bg_pad)

    q = qkv[:, 0:D].reshape(T, H, HD).transpose(1, 0, 2)
    k = qkv[:, D:2 * D].reshape(T, H, HD).transpose(1, 0, 2)
    v = qkv[:, 2 * D:3 * D].reshape(T, H, HD).transpose(1, 0, 2)

    out3 = pl.pallas_call(
        _attn_kernel,
        grid=(H, T // TQ),
        in_specs=[
            pl.BlockSpec((1, TQ, HD), lambda h, qc: (h, qc, 0)),
            pl.BlockSpec((1, T, HD), lambda h, qc: (h, 0, 0)),
            pl.BlockSpec((1, T, HD), lambda h, qc: (h, 0, 0)),
            pl.BlockSpec((TQ, 128), lambda h, qc: (qc, 0)),
            pl.BlockSpec((HD, G * HD), lambda h, qc: (0, 0)),
            pl.BlockSpec((HD, G * HD), lambda h, qc: (0, 0)),
        ],
        out_specs=pl.BlockSpec((1, TQ, HD), lambda h, qc: (h, qc, 0)),
        out_shape=jax.ShapeDtypeStruct((H, T, HD), jnp.float32),
        scratch_shapes=[
            pltpu.VMEM((NB, HD), jnp.float32),
            pltpu.VMEM((NB, HD), jnp.float32),
        ],
        interpret=_INTERPRET,
    )(q, k, v, gates, W_ck, W_cv)

    merged = out3.transpose(1, 0, 2).reshape(T, D)

    out = pl.pallas_call(
        _proj_kernel,
        grid=(T // TQ,),
        in_specs=[
            pl.BlockSpec((TQ, D), lambda i: (i, 0)),
            pl.BlockSpec((D, D), lambda i: (0, 0)),
        ],
        out_specs=pl.BlockSpec((TQ, D), lambda i: (i, 0)),
        out_shape=jax.ShapeDtypeStruct((T, D), jnp.float32),
        interpret=_INTERPRET,
    )(merged, W_o)

    return out.reshape(B, T, D)


# speedup vs baseline: 101.8535x; 1.0048x over previous
import math

import jax
import jax.numpy as jnp
from jax.experimental import pallas as pl
from jax.experimental.pallas import tpu as pltpu

B, T, D = 1, 2048, 768
H, HD = 12, 64
G, R, WIN = 16, 4, 64
NB = T // G
SCALE = math.sqrt(HD)
HALF = WIN // 2
TQ = 256
TK = 256
BAND = 2 * TK
NEG_TOPK = -1.0e30
NEG_TAKEN = -3.0e38
NEG_MASK = -3.0e38

_INTERPRET = False


def _safe_sm(s):
    m = jnp.max(s, axis=-1, keepdims=True)
    m = jnp.where(m > -jnp.inf, m, 0.0)
    e = jnp.exp(s - m)
    z = jnp.sum(e, axis=-1, keepdims=True)
    return e / jnp.where(z == 0.0, 1.0, z)


def _qkv_gates_kernel(x_ref, wq_ref, wk_ref, wv_ref, wg_ref, bg_ref,
                      q_ref, k_ref, v_ref, g_ref):
    xb = x_ref[...]
    qm = jnp.zeros((TQ, HD), jnp.float32)
    for h in range(H):
        qh = jax.lax.dot_general(xb, wq_ref[h], (((1,), (1,)), ((), ())),
                                 preferred_element_type=jnp.float32)
        q_ref[h] = qh
        qm = qm + qh
        k_ref[h] = jax.lax.dot_general(xb, wk_ref[h], (((1,), (1,)), ((), ())),
                                       preferred_element_type=jnp.float32)
        v_ref[h] = jax.lax.dot_general(xb, wv_ref[h], (((1,), (1,)), ((), ())),
                                       preferred_element_type=jnp.float32)
    qm = qm * (1.0 / H)
    glog = jax.lax.dot_general(qm, wg_ref[...], (((1,), (1,)), ((), ())),
                               preferred_element_type=jnp.float32)
    glog = glog + bg_ref[...]
    m = jnp.max(glog, axis=-1, keepdims=True)
    e = jnp.exp(glog - m)
    g_ref[...] = e / jnp.sum(e, axis=-1, keepdims=True)


def _attn_kernel(q_ref, k_ref, v_ref, g_ref, wck_ref, wcv_ref, out_ref,
                 kc_s, vc_s, selm_s):
    qc = pl.program_id(1)

    @pl.when(qc == 0)
    def _compress():
        k3 = k_ref[0].reshape(NB, G, HD)
        v3 = v_ref[0].reshape(NB, G, HD)
        kc = jnp.zeros((NB, HD), jnp.float32)
        vc = jnp.zeros((NB, HD), jnp.float32)
        for g in range(G):
            wck_g = wck_ref[:, g * HD:(g + 1) * HD]
            wcv_g = wcv_ref[:, g * HD:(g + 1) * HD]
            kc = kc + jax.lax.dot_general(k3[:, g, :], wck_g,
                                          (((1,), (1,)), ((), ())),
                                          preferred_element_type=jnp.float32)
            vc = vc + jax.lax.dot_general(v3[:, g, :], wcv_g,
                                          (((1,), (1,)), ((), ())),
                                          preferred_element_type=jnp.float32)
        kc_s[...] = kc
        vc_s[...] = vc

    q = q_ref[0]
    qs = q * (1.0 / SCALE)
    sc = jax.lax.dot_general(qs, kc_s[...], (((1,), (1,)), ((), ())),
                             preferred_element_type=jnp.float32)
    bi = jax.lax.broadcasted_iota(jnp.int32, (TQ, NB), 1)
    ri = jax.lax.broadcasted_iota(jnp.int32, (TQ, NB), 0) + qc * TQ
    valid = ri >= (bi + 1) * G
    out_c = jax.lax.dot_general(_safe_sm(jnp.where(valid, sc, -jnp.inf)),
                                vc_s[...], (((1,), (0,)), ((), ())),
                                preferred_element_type=jnp.float32)
    s = jnp.where(valid, sc, NEG_TOPK)
    bmask = jnp.zeros((TQ, NB), jnp.float32)
    for _ in range(R):
        m = jnp.max(s, axis=-1, keepdims=True)
        idx = jnp.min(jnp.where(s == m, bi, NB), axis=-1, keepdims=True)
        hit = bi == idx
        bmask = jnp.where(hit, 1.0, bmask)
        s = jnp.where(hit, NEG_TAKEN, s)
    en = jax.lax.broadcasted_iota(jnp.int32, (NB, T), 0)
    es = jax.lax.broadcasted_iota(jnp.int32, (NB, T), 1)
    E = jnp.where(es // G == en, 1.0, 0.0)
    selm_s[...] = jax.lax.dot_general(bmask, E, (((1,), (0,)), ((), ())),
                                      preferred_element_type=jnp.float32)
    ti_k = jax.lax.broadcasted_iota(jnp.int32, (TQ, TK), 1)
    tr_k = jax.lax.broadcasted_iota(jnp.int32, (TQ, TK), 0) + qc * TQ

    def body(j, carry):
        m_o, z_o, u_o = carry
        k_j = k_ref[0, pl.ds(j * TK, TK), :]
        v_j = v_ref[0, pl.ds(j * TK, TK), :]
        s_j = jax.lax.dot_general(qs, k_j, (((1,), (1,)), ((), ())),
                                  preferred_element_type=jnp.float32)
        sm_j = selm_s[:, pl.ds(j * TK, TK)] > 0.5
        mask = sm_j & ((qc != 0) | (ti_k <= tr_k))
        s_j = jnp.where(mask, s_j, NEG_MASK)
        m_n = jnp.maximum(m_o, jnp.max(s_j, axis=-1, keepdims=True))
        e_j = jnp.exp(s_j - m_n)
        corr = jnp.exp(m_o - m_n)
        z_n = z_o * corr + jnp.sum(e_j, axis=-1, keepdims=True)
        u_n = u_o * corr + jax.lax.dot_general(
            e_j, v_j, (((1,), (0,)), ((), ())),
            preferred_element_type=jnp.float32)
        return m_n, z_n, u_n

    m0 = jnp.full((TQ, 1), -1.0e30, jnp.float32)
    z0 = jnp.zeros((TQ, 1), jnp.float32)
    u0 = jnp.zeros((TQ, HD), jnp.float32)
    m_s, z_s, u_s = jax.lax.fori_loop(0, qc + 1, body, (m0, z0, u0))
    start = jnp.maximum(qc - 1, 0) * TK
    k_b = k_ref[0, pl.ds(start, BAND), :]
    v_b = v_ref[0, pl.ds(start, BAND), :]
    s_b = jax.lax.dot_general(qs, k_b, (((1,), (1,)), ((), ())),
                              preferred_element_type=jnp.float32)
    ci = jax.lax.broadcasted_iota(jnp.int32, (TQ, BAND), 1) + start
    rw = jax.lax.broadcasted_iota(jnp.int32, (TQ, BAND), 0) + qc * TQ
    s_b = jnp.where((ci <= rw) & (ci >= rw - HALF), s_b, NEG_MASK)
    m_l = jnp.max(s_b, axis=-1, keepdims=True)
    e_b = jnp.exp(s_b - m_l)
    z_l = jnp.sum(e_b, axis=-1, keepdims=True)
    g = g_ref[...]
    alpha = g[:, 0:1]
    beta = g[:, 1:2] / jnp.where(z_s == 0.0, 1.0, z_s)
    gamma = g[:, 2:3] / z_l
    out_l = jax.lax.dot_general(e_b, v_b, (((1,), (0,)), ((), ())),
                                preferred_element_type=jnp.float32)
    out_ref[0] = alpha * out_c + beta * u_s + gamma * out_l


def _proj_kernel(x_ref, wo_ref, o_ref):
    acc = jnp.zeros((TQ, D), jnp.float32)
    for h in range(H):
        acc = acc + jax.lax.dot_general(x_ref[h], wo_ref[h],
                                        (((1,), (0,)), ((), ())),
                                        preferred_element_type=jnp.float32)
    o_ref[...] = acc


def kernel(x, W_q, W_k, W_v, W_o, W_ck, W_cv, W_g, b_g):
    x2d = x.reshape(T, D)
    wq3 = W_q.reshape(H, HD, D)
    wk3 = W_k.reshape(H, HD, D)
    wv3 = W_v.reshape(H, HD, D)
    wo3 = W_o.T.reshape(H, HD, D)
    Wg_pad = jnp.zeros((128, HD), jnp.float32).at[:3].set(W_g)
    bg_pad = jnp.full((1, 128), -1.0e30, jnp.float32).at[0, :3].set(b_g)

    q, k, v, gates = pl.pallas_call(
        _qkv_gates_kernel,
        grid=(T // TQ,),
        in_specs=[
            pl.BlockSpec((TQ, D), lambda i: (i, 0)),
            pl.BlockSpec((H, HD, D), lambda i: (0, 0, 0)),
            pl.BlockSpec((H, HD, D), lambda i: (0, 0, 0)),
            pl.BlockSpec((H, HD, D), lambda i: (0, 0, 0)),
            pl.BlockSpec((128, HD), lambda i: (0, 0)),
            pl.BlockSpec((1, 128), lambda i: (0, 0)),
        ],
        out_specs=[
            pl.BlockSpec((H, TQ, HD), lambda i: (0, i, 0)),
            pl.BlockSpec((H, TQ, HD), lambda i: (0, i, 0)),
            pl.BlockSpec((H, TQ, HD), lambda i: (0, i, 0)),
            pl.BlockSpec((TQ, 128), lambda i: (i, 0)),
        ],
        out_shape=[
            jax.ShapeDtypeStruct((H, T, HD), jnp.float32),
            jax.ShapeDtypeStruct((H, T, HD), jnp.float32),
            jax.ShapeDtypeStruct((H, T, HD), jnp.float32),
            jax.ShapeDtypeStruct((T, 128), jnp.float32),
        ],
        interpret=_INTERPRET,
    )(x2d, wq3, wk3, wv3, Wg_pad, bg_pad)

    out3 = pl.pallas_call(
        _attn_kernel,
        grid=(H, T // TQ),
        in_specs=[
            pl.BlockSpec((1, TQ, HD), lambda h, qc: (h, qc, 0)),
            pl.BlockSpec((1, T, HD), lambda h, qc: (h, 0, 0)),
            pl.BlockSpec((1, T, HD), lambda h, qc: (h, 0, 0)),
            pl.BlockSpec((TQ, 128), lambda h, qc: (qc, 0)),
            pl.BlockSpec((HD, G * HD), lambda h, qc: (0, 0)),
            pl.BlockSpec((HD, G * HD), lambda h, qc: (0, 0)),
        ],
        out_specs=pl.BlockSpec((1, TQ, HD), lambda h, qc: (h, qc, 0)),
        out_shape=jax.ShapeDtypeStruct((H, T, HD), jnp.float32),
        scratch_shapes=[
            pltpu.VMEM((NB, HD), jnp.float32),
            pltpu.VMEM((NB, HD), jnp.float32),
            pltpu.VMEM((TQ, T), jnp.float32),
        ],
        interpret=_INTERPRET,
    )(q, k, v, gates, W_ck, W_cv)

    out = pl.pallas_call(
        _proj_kernel,
        grid=(T // TQ,),
        in_specs=[
            pl.BlockSpec((H, TQ, HD), lambda i: (0, i, 0)),
            pl.BlockSpec((H, HD, D), lambda i: (0, 0, 0)),
        ],
        out_specs=pl.BlockSpec((TQ, D), lambda i: (i, 0)),
        out_shape=jax.ShapeDtypeStruct((T, D), jnp.float32),
        interpret=_INTERPRET,
    )(out3, wo3)

    return out.reshape(B, T, D)


# speedup vs baseline: 115.3574x; 1.1326x over previous
import math

import jax
import jax.numpy as jnp
from jax.experimental import pallas as pl
from jax.experimental.pallas import tpu as pltpu

B, T, D = 1, 2048, 768
H, HD = 12, 64
G, R, WIN = 16, 4, 64
NB = T // G
SCALE = math.sqrt(HD)
HALF = WIN // 2
TQ = 256
TK = 256
BAND = 2 * TK
NEG_TOPK = -1.0e30
NEG_TAKEN = -3.0e38
NEG_MASK = -3.0e38
BIG = 2.0 ** 100

_INTERPRET = False


def _safe_sm(s):
    m = jnp.max(s, axis=-1, keepdims=True)
    m = jnp.where(m > -jnp.inf, m, 0.0)
    e = jnp.exp(s - m)
    z = jnp.sum(e, axis=-1, keepdims=True)
    return e / jnp.where(z == 0.0, 1.0, z)


def _qkv_gates_kernel(x_ref, w_ref, wg_ref, bg_ref,
                      q_ref, k_ref, v_ref, g_ref):
    qkv = jax.lax.dot_general(x_ref[...], w_ref[...], (((1,), (0,)), ((), ())),
                              preferred_element_type=jnp.float32)
    qm = jnp.zeros((TQ, HD), jnp.float32)
    for h in range(H):
        qh = qkv[:, h * HD:(h + 1) * HD]
        q_ref[h] = qh
        qm = qm + qh
        k_ref[h] = qkv[:, D + h * HD:D + (h + 1) * HD]
        v_ref[h] = qkv[:, 2 * D + h * HD:2 * D + (h + 1) * HD]
    qm = qm * (1.0 / H)
    glog = jax.lax.dot_general(qm, wg_ref[...], (((1,), (1,)), ((), ())),
                               preferred_element_type=jnp.float32)
    glog = glog + bg_ref[...]
    m = jnp.max(glog, axis=-1, keepdims=True)
    e = jnp.exp(glog - m)
    g_ref[...] = e / jnp.sum(e, axis=-1, keepdims=True)


def _attn_kernel(q_ref, k_ref, v_ref, g_ref, wck_ref, wcv_ref, e_ref,
                 out_ref, kc_s, vc_s, s_s):
    hh = pl.program_id(0)
    qc = pl.program_id(1)

    @pl.when((hh == 0) & (qc == 0))
    def _init():
        s_s[...] = jnp.zeros((TQ, T), jnp.float32)

    @pl.when(qc == 0)
    def _compress():
        k3 = k_ref[0].reshape(NB, G, HD)
        v3 = v_ref[0].reshape(NB, G, HD)
        kc = jnp.zeros((NB, HD), jnp.float32)
        vc = jnp.zeros((NB, HD), jnp.float32)
        for g in range(G):
            wck_g = wck_ref[:, g * HD:(g + 1) * HD]
            wcv_g = wcv_ref[:, g * HD:(g + 1) * HD]
            kc = kc + jax.lax.dot_general(k3[:, g, :], wck_g,
                                          (((1,), (1,)), ((), ())),
                                          preferred_element_type=jnp.float32)
            vc = vc + jax.lax.dot_general(v3[:, g, :], wcv_g,
                                          (((1,), (1,)), ((), ())),
                                          preferred_element_type=jnp.float32)
        kc_s[...] = kc
        vc_s[...] = vc

    q = q_ref[0]
    qs = q * (1.0 / SCALE)
    sc = jax.lax.dot_general(qs, kc_s[...], (((1,), (1,)), ((), ())),
                             preferred_element_type=jnp.float32)
    bi = jax.lax.broadcasted_iota(jnp.int32, (TQ, NB), 1)
    ri = jax.lax.broadcasted_iota(jnp.int32, (TQ, NB), 0) + qc * TQ
    valid = ri >= (bi + 1) * G
    out_c = jax.lax.dot_general(_safe_sm(jnp.where(valid, sc, -jnp.inf)),
                                vc_s[...], (((1,), (0,)), ((), ())),
                                preferred_element_type=jnp.float32)
    s = jnp.where(valid, sc, NEG_TOPK)
    bmask = jnp.zeros((TQ, NB), jnp.float32)
    for _ in range(R):
        m = jnp.max(s, axis=-1, keepdims=True)
        idx = jnp.min(jnp.where(s == m, bi, NB), axis=-1, keepdims=True)
        hit = bi == idx
        bmask = jnp.where(hit, 1.0, bmask)
        s = jnp.where(hit, NEG_TAKEN, s)
    for j in range(T // TK):
        @pl.when(j <= qc)
        def _chunk(j=j):
            s_s[:, j * TK:(j + 1) * TK] = jax.lax.dot_general(
                qs, k_ref[0, j * TK:(j + 1) * TK, :], (((1,), (1,)), ((), ())),
                preferred_element_type=jnp.float32)
    @pl.when(qc == 0)
    def _fill_causal():
        ti = jax.lax.broadcasted_iota(jnp.int32, (TQ, TK), 1)
        tr = jax.lax.broadcasted_iota(jnp.int32, (TQ, TK), 0)
        s_s[:, 0:TK] = jnp.where(ti <= tr, s_s[:, 0:TK], -BIG)

    bias = jax.lax.dot_general(bmask, e_ref[...], (((1,), (0,)), ((), ())),
                               preferred_element_type=jnp.float32) - BIG
    sb = s_s[...] + bias
    m_sel = jnp.max(sb, axis=-1, keepdims=True)
    e_sel = jnp.exp(sb - m_sel)
    z_s = jnp.sum(e_sel, axis=-1, keepdims=True)
    u_s = jax.lax.dot_general(e_sel, v_ref[0], (((1,), (0,)), ((), ())),
                              preferred_element_type=jnp.float32)
    start = jnp.maximum(qc - 1, 0) * TK
    k_b = k_ref[0, pl.ds(start, BAND), :]
    v_b = v_ref[0, pl.ds(start, BAND), :]
    s_b = jax.lax.dot_general(qs, k_b, (((1,), (1,)), ((), ())),
                              preferred_element_type=jnp.float32)
    ci = jax.lax.broadcasted_iota(jnp.int32, (TQ, BAND), 1) + start
    rw = jax.lax.broadcasted_iota(jnp.int32, (TQ, BAND), 0) + qc * TQ
    s_b = jnp.where((ci <= rw) & (ci >= rw - HALF), s_b, NEG_MASK)
    m_l = jnp.max(s_b, axis=-1, keepdims=True)
    e_b = jnp.exp(s_b - m_l)
    z_l = jnp.sum(e_b, axis=-1, keepdims=True)
    out_l = jax.lax.dot_general(e_b, v_b, (((1,), (0,)), ((), ())),
                                preferred_element_type=jnp.float32)
    g = g_ref[...]
    alpha = g[:, 0:1]
    beta = g[:, 1:2] / jnp.where(z_s == 0.0, 1.0, z_s)
    gamma = g[:, 2:3] / z_l
    out_ref[0] = alpha * out_c + beta * u_s + gamma * out_l


def _proj_kernel(x_ref, wo_ref, o_ref, m_s):
    for h in range(H):
        m_s[:, h * HD:(h + 1) * HD] = x_ref[h]
    o_ref[...] = jax.lax.dot_general(m_s[...], wo_ref[...],
                                     (((1,), (1,)), ((), ())),
                                     preferred_element_type=jnp.float32)


def kernel(x, W_q, W_k, W_v, W_o, W_ck, W_cv, W_g, b_g):
    x2d = x.reshape(T, D)
    W_qkv = jnp.concatenate([W_q, W_k, W_v], axis=0).T
    Wg_pad = jnp.zeros((128, HD), jnp.float32).at[:3].set(W_g)
    bg_pad = jnp.full((1, 128), -1.0e30, jnp.float32).at[0, :3].set(b_g)
    E = jnp.where(jnp.arange(T)[None, :] // G == jnp.arange(NB)[:, None],
                  BIG, 0.0).astype(jnp.float32)

    q, k, v, gates = pl.pallas_call(
        _qkv_gates_kernel,
        grid=(T // TQ,),
        in_specs=[
            pl.BlockSpec((TQ, D), lambda i: (i, 0)),
            pl.BlockSpec((D, 3 * D), lambda i: (0, 0)),
            pl.BlockSpec((128, HD), lambda i: (0, 0)),
            pl.BlockSpec((1, 128), lambda i: (0, 0)),
        ],
        out_specs=[
            pl.BlockSpec((H, TQ, HD), lambda i: (0, i, 0)),
            pl.BlockSpec((H, TQ, HD), lambda i: (0, i, 0)),
            pl.BlockSpec((H, TQ, HD), lambda i: (0, i, 0)),
            pl.BlockSpec((TQ, 128), lambda i: (i, 0)),
        ],
        out_shape=[
            jax.ShapeDtypeStruct((H, T, HD), jnp.float32),
            jax.ShapeDtypeStruct((H, T, HD), jnp.float32),
            jax.ShapeDtypeStruct((H, T, HD), jnp.float32),
            jax.ShapeDtypeStruct((T, 128), jnp.float32),
        ],
        interpret=_INTERPRET,
    )(x2d, W_qkv, Wg_pad, bg_pad)

    out3 = pl.pallas_call(
        _attn_kernel,
        grid=(H, T // TQ),
        in_specs=[
            pl.BlockSpec((1, TQ, HD), lambda h, qc: (h, qc, 0)),
            pl.BlockSpec((1, T, HD), lambda h, qc: (h, 0, 0)),
            pl.BlockSpec((1, T, HD), lambda h, qc: (h, 0, 0)),
            pl.BlockSpec((TQ, 128), lambda h, qc: (qc, 0)),
            pl.BlockSpec((HD, G * HD), lambda h, qc: (0, 0)),
            pl.BlockSpec((HD, G * HD), lambda h, qc: (0, 0)),
            pl.BlockSpec((NB, T), lambda h, qc: (0, 0)),
        ],
        out_specs=pl.BlockSpec((1, TQ, HD), lambda h, qc: (h, qc, 0)),
        out_shape=jax.ShapeDtypeStruct((H, T, HD), jnp.float32),
        scratch_shapes=[
            pltpu.VMEM((NB, HD), jnp.float32),
            pltpu.VMEM((NB, HD), jnp.float32),
            pltpu.VMEM((TQ, T), jnp.float32),
        ],
        interpret=_INTERPRET,
    )(q, k, v, gates, W_ck, W_cv, E)

    out = pl.pallas_call(
        _proj_kernel,
        grid=(T // TQ,),
        in_specs=[
            pl.BlockSpec((H, TQ, HD), lambda i: (0, i, 0)),
            pl.BlockSpec((D, D), lambda i: (0, 0)),
        ],
        out_specs=pl.BlockSpec((TQ, D), lambda i: (i, 0)),
        out_shape=jax.ShapeDtypeStruct((T, D), jnp.float32),
        scratch_shapes=[pltpu.VMEM((TQ, D), jnp.float32)],
        interpret=_INTERPRET,
    )(out3, W_o)

    return out.reshape(B, T, D)


# speedup vs baseline: 142.4134x; 1.2345x over previous
import math

import jax
import jax.numpy as jnp
from jax.experimental import pallas as pl
from jax.experimental.pallas import tpu as pltpu

B, T, D = 1, 2048, 768
H, HD = 12, 64
G, R, WIN = 16, 4, 64
NB = T // G
SCALE = math.sqrt(HD)
HALF = WIN // 2
TQ = 256
TK = 256
BAND = 2 * TK
NEG_TOPK = -1.0e30
NEG_TAKEN = -3.0e38
NEG_MASK = -3.0e38
BIG = 2.0 ** 100

_INTERPRET = False


def _safe_sm(s):
    m = jnp.max(s, axis=-1, keepdims=True)
    m = jnp.where(m > -jnp.inf, m, 0.0)
    e = jnp.exp(s - m)
    z = jnp.sum(e, axis=-1, keepdims=True)
    return e / jnp.where(z == 0.0, 1.0, z)


def _qkv_gates_kernel(x_ref, w_ref, wg_ref, bg_ref,
                      q_ref, k_ref, v_ref, g_ref):
    qkv = jax.lax.dot_general(x_ref[...], w_ref[...], (((1,), (0,)), ((), ())),
                              preferred_element_type=jnp.float32)
    qm = jnp.zeros((TQ, HD), jnp.float32)
    for h in range(H):
        qh = qkv[:, h * HD:(h + 1) * HD]
        q_ref[h] = qh
        qm = qm + qh
        k_ref[h] = qkv[:, D + h * HD:D + (h + 1) * HD]
        v_ref[h] = qkv[:, 2 * D + h * HD:2 * D + (h + 1) * HD]
    qm = qm * (1.0 / H)
    glog = jax.lax.dot_general(qm, wg_ref[...], (((1,), (1,)), ((), ())),
                               preferred_element_type=jnp.float32)
    glog = glog + bg_ref[...]
    m = jnp.max(glog, axis=-1, keepdims=True)
    e = jnp.exp(glog - m)
    g_ref[...] = e / jnp.sum(e, axis=-1, keepdims=True)


def _attn_kernel(q_ref, k_ref, v_ref, g_ref, wck_ref, wcv_ref, e_ref,
                 out_ref, kc_s, vc_s):
    qc = pl.program_id(1)

    @pl.when(qc == 0)
    def _compress():
        k3 = k_ref[0].reshape(NB, G, HD)
        v3 = v_ref[0].reshape(NB, G, HD)
        kc = jnp.zeros((NB, HD), jnp.float32)
        vc = jnp.zeros((NB, HD), jnp.float32)
        for g in range(G):
            wck_g = wck_ref[:, g * HD:(g + 1) * HD]
            wcv_g = wcv_ref[:, g * HD:(g + 1) * HD]
            kc = kc + jax.lax.dot_general(k3[:, g, :], wck_g,
                                          (((1,), (1,)), ((), ())),
                                          preferred_element_type=jnp.float32)
            vc = vc + jax.lax.dot_general(v3[:, g, :], wcv_g,
                                          (((1,), (1,)), ((), ())),
                                          preferred_element_type=jnp.float32)
        kc_s[...] = kc
        vc_s[...] = vc

    q = q_ref[0]
    qs = q * (1.0 / SCALE)
    sc = jax.lax.dot_general(qs, kc_s[...], (((1,), (1,)), ((), ())),
                             preferred_element_type=jnp.float32)
    bi = jax.lax.broadcasted_iota(jnp.int32, (TQ, NB), 1)
    ri = jax.lax.broadcasted_iota(jnp.int32, (TQ, NB), 0) + qc * TQ
    valid = ri >= (bi + 1) * G
    out_c = jax.lax.dot_general(_safe_sm(jnp.where(valid, sc, -jnp.inf)),
                                vc_s[...], (((1,), (0,)), ((), ())),
                                preferred_element_type=jnp.float32)
    s = jnp.where(valid, sc, NEG_TOPK)
    bmask = jnp.zeros((TQ, NB), jnp.float32)
    for _ in range(R):
        m = jnp.max(s, axis=-1, keepdims=True)
        idx = jnp.min(jnp.where(s == m, bi, NB), axis=-1, keepdims=True)
        hit = bi == idx
        bmask = jnp.where(hit, 1.0, bmask)
        s = jnp.where(hit, NEG_TAKEN, s)
    start = jnp.maximum(qc - 1, 0) * TK
    k_b = k_ref[0, pl.ds(start, BAND), :]
    v_b = v_ref[0, pl.ds(start, BAND), :]
    s_b = jax.lax.dot_general(qs, k_b, (((1,), (1,)), ((), ())),
                              preferred_element_type=jnp.float32)
    ci = jax.lax.broadcasted_iota(jnp.int32, (TQ, BAND), 1) + start
    rw = jax.lax.broadcasted_iota(jnp.int32, (TQ, BAND), 0) + qc * TQ
    s_b = jnp.where((ci <= rw) & (ci >= rw - HALF), s_b, NEG_MASK)
    m_l = jnp.max(s_b, axis=-1, keepdims=True)
    e_b = jnp.exp(s_b - m_l)
    z_l = jnp.sum(e_b, axis=-1, keepdims=True)
    out_l = jax.lax.dot_general(e_b, v_b, (((1,), (0,)), ((), ())),
                                preferred_element_type=jnp.float32)
    g = g_ref[...]
    alpha = g[:, 0:1]
    g_beta = g[:, 1:2]
    gamma_out_l = (g[:, 2:3] / z_l) * out_l
    rest = alpha * out_c + gamma_out_l

    def _selected(width, fill_causal):
        kw = k_ref[0, 0:width, :]
        vw = v_ref[0, 0:width, :]
        s_w = jax.lax.dot_general(qs, kw, (((1,), (1,)), ((), ())),
                                  preferred_element_type=jnp.float32)
        bias = jax.lax.dot_general(bmask, e_ref[:, 0:width],
                                   (((1,), (0,)), ((), ())),
                                   preferred_element_type=jnp.float32) - BIG
        sb = s_w + bias
        if fill_causal:
            ti = jax.lax.broadcasted_iota(jnp.int32, (TQ, width), 1)
            tr = jax.lax.broadcasted_iota(jnp.int32, (TQ, width), 0)
            sb = jnp.where(ti <= tr, sb, -BIG)
        m_sel = jnp.max(sb, axis=-1, keepdims=True)
        e_sel = jnp.exp(sb - m_sel)
        z_s = jnp.sum(e_sel, axis=-1, keepdims=True)
        u_s = jax.lax.dot_general(e_sel, vw, (((1,), (0,)), ((), ())),
                                  preferred_element_type=jnp.float32)
        beta = g_beta / jnp.where(z_s == 0.0, 1.0, z_s)
        out_ref[0] = rest + beta * u_s

    @pl.when(qc == 0)
    def _b0():
        _selected(TK, True)

    @pl.when((qc >= 1) & (qc <= 3))
    def _b1():
        _selected(4 * TK, False)

    @pl.when(qc >= 4)
    def _b2():
        _selected(T, False)


def _proj_kernel(x_ref, wo_ref, o_ref, m_s):
    for h in range(H):
        m_s[:, h * HD:(h + 1) * HD] = x_ref[h]
    o_ref[...] = jax.lax.dot_general(m_s[...], wo_ref[...],
                                     (((1,), (1,)), ((), ())),
                                     preferred_element_type=jnp.float32)


def kernel(x, W_q, W_k, W_v, W_o, W_ck, W_cv, W_g, b_g):
    x2d = x.reshape(T, D)
    W_qkv = jnp.concatenate([W_q, W_k, W_v], axis=0).T
    Wg_pad = jnp.zeros((128, HD), jnp.float32).at[:3].set(W_g)
    bg_pad = jnp.full((1, 128), -1.0e30, jnp.float32).at[0, :3].set(b_g)
    E = jnp.where(jnp.arange(T)[None, :] // G == jnp.arange(NB)[:, None],
                  BIG, 0.0).astype(jnp.float32)

    q, k, v, gates = pl.pallas_call(
        _qkv_gates_kernel,
        grid=(T // TQ,),
        in_specs=[
            pl.BlockSpec((TQ, D), lambda i: (i, 0)),
            pl.BlockSpec((D, 3 * D), lambda i: (0, 0)),
            pl.BlockSpec((128, HD), lambda i: (0, 0)),
            pl.BlockSpec((1, 128), lambda i: (0, 0)),
        ],
        out_specs=[
            pl.BlockSpec((H, TQ, HD), lambda i: (0, i, 0)),
            pl.BlockSpec((H, TQ, HD), lambda i: (0, i, 0)),
            pl.BlockSpec((H, TQ, HD), lambda i: (0, i, 0)),
            pl.BlockSpec((TQ, 128), lambda i: (i, 0)),
        ],
        out_shape=[
            jax.ShapeDtypeStruct((H, T, HD), jnp.float32),
            jax.ShapeDtypeStruct((H, T, HD), jnp.float32),
            jax.ShapeDtypeStruct((H, T, HD), jnp.float32),
            jax.ShapeDtypeStruct((T, 128), jnp.float32),
        ],
        interpret=_INTERPRET,
    )(x2d, W_qkv, Wg_pad, bg_pad)

    out3 = pl.pallas_call(
        _attn_kernel,
        grid=(H, T // TQ),
        in_specs=[
            pl.BlockSpec((1, TQ, HD), lambda h, qc: (h, qc, 0)),
            pl.BlockSpec((1, T, HD), lambda h, qc: (h, 0, 0)),
            pl.BlockSpec((1, T, HD), lambda h, qc: (h, 0, 0)),
            pl.BlockSpec((TQ, 128), lambda h, qc: (qc, 0)),
            pl.BlockSpec((HD, G * HD), lambda h, qc: (0, 0)),
            pl.BlockSpec((HD, G * HD), lambda h, qc: (0, 0)),
            pl.BlockSpec((NB, T), lambda h, qc: (0, 0)),
        ],
        out_specs=pl.BlockSpec((1, TQ, HD), lambda h, qc: (h, qc, 0)),
        out_shape=jax.ShapeDtypeStruct((H, T, HD), jnp.float32),
        scratch_shapes=[
            pltpu.VMEM((NB, HD), jnp.float32),
            pltpu.VMEM((NB, HD), jnp.float32),
        ],
        interpret=_INTERPRET,
    )(q, k, v, gates, W_ck, W_cv, E)

    out = pl.pallas_call(
        _proj_kernel,
        grid=(T // TQ,),
        in_specs=[
            pl.BlockSpec((H, TQ, HD), lambda i: (0, i, 0)),
            pl.BlockSpec((D, D), lambda i: (0, 0)),
        ],
        out_specs=pl.BlockSpec((TQ, D), lambda i: (i, 0)),
        out_shape=jax.ShapeDtypeStruct((T, D), jnp.float32),
        scratch_shapes=[pltpu.VMEM((TQ, D), jnp.float32)],
        interpret=_INTERPRET,
    )(out3, W_o)

    return out.reshape(B, T, D)
